# baseline (speedup 1.0000x reference)
import jax
import jax.numpy as jnp
from jax import lax
from jax.experimental import pallas as pl
from jax.experimental.pallas import tpu as pltpu
from jax.experimental.pallas import tpu_sc as plsc

N_POINTS = 100000
N_BINS = 512
NC = 2
NS = 16
L = 16

CHUNK1 = 6272
CHUNK1_LAST = N_POINTS - (NS - 1) * CHUNK1
P1_CHUNKS = ((768, 8), (2816, 8), (2688, 8))
P1_CHUNKS_LAST = ((640, 8), (2640, 5), (2640, 5))
HALF1 = CHUNK1 // 2
HALF1_LAST = CHUNK1_LAST // 2
NBP = 576
ACC = 4 * NBP
PLY = NBP
COLCH = ACC // NS


def _rsqrt(x):
    xb = plsc.bitcast(x, jnp.int32)
    y = plsc.bitcast(jnp.int32(0x5F3759DF) - lax.shift_right_logical(xb, 1),
                     jnp.float32)
    hx = x * 0.5
    for _ in range(3):
        y = y * (1.5 - hx * y * y)
    return y


def _floxels_kernel(fx_hbm, fy_hbm, fz_hbm, clus_hbm, out_hbm,
                    fx_v, fy_v, fz_v, clus_v, hist_v, colchunk_v,
                    combchunk_v, out_v, sem_a, sem_b, part_sh, comb_sh):
    c = lax.axis_index("c")
    s = lax.axis_index("s")
    zeros16 = jnp.zeros((L,), jnp.float32)
    ones16 = jnp.ones((L,), jnp.float32)

    def fire(base, npts, lb, sem):
        return [
            pltpu.async_copy(fx_hbm.at[pl.ds(base, npts)],
                             fx_v.at[pl.ds(lb, npts)], sem),
            pltpu.async_copy(fy_hbm.at[pl.ds(base, npts)],
                             fy_v.at[pl.ds(lb, npts)], sem),
            pltpu.async_copy(fz_hbm.at[pl.ds(base, npts)],
                             fz_v.at[pl.ds(lb, npts)], sem),
            pltpu.async_copy(clus_hbm.at[pl.ds(base, npts)],
                             clus_v.at[pl.ds(lb, npts)], sem),
        ]

    def p1_loop(lb, npts, unroll):
        @plsc.parallel_loop(lb, lb + npts, step=L, unroll=unroll)
        def _p1(b):
            cl = clus_v[pl.ds(b, L)]
            plsc.addupdate_scatter(hist_v, [cl], fx_v[pl.ds(b, L)])
            plsc.addupdate_scatter(hist_v, [cl + PLY], fy_v[pl.ds(b, L)])
            plsc.addupdate_scatter(hist_v, [cl + 2 * PLY], fz_v[pl.ds(b, L)])
            plsc.addupdate_scatter(hist_v, [cl + 3 * PLY], ones16)

    def phase1(base, chunks):
        (sz0, u0), (sz1, u1), (sz2, u2) = chunks
        cps0 = fire(base, sz0, 0, sem_a)
        cps1 = fire(base + sz0, sz1, sz0, sem_b)

        @plsc.parallel_loop(0, ACC, step=L, unroll=4)
        def _zero(b):
            hist_v[pl.ds(b, L)] = zeros16

        for cp in cps0:
            cp.wait()
        cps2 = fire(base + sz0 + sz1, sz2, sz0 + sz1, sem_a)
        p1_loop(0, sz0, u0)
        for cp in cps1:
            cp.wait()
        p1_loop(sz0, sz1, u1)
        for cp in cps2:
            cp.wait()
        p1_loop(sz0 + sz1, sz2, u2)

    with jax.named_scope("p1"):
        pl.when(s < NS - 1)(lambda: phase1(s * CHUNK1, P1_CHUNKS))
        pl.when(s == NS - 1)(lambda: phase1((NS - 1) * CHUNK1, P1_CHUNKS_LAST))

    scope2 = jax.named_scope("p2")
    scope2.__enter__()
    pltpu.sync_copy(hist_v, part_sh.at[pl.ds(s * ACC, ACC)])
    plsc.subcore_barrier()
    cps = [
        pltpu.async_copy(part_sh.at[pl.ds(l * ACC + s * COLCH, COLCH)],
                         colchunk_v.at[pl.ds(l * COLCH, COLCH)], sem_a)
        for l in range(NS)
    ]
    for cp in cps:
        cp.wait()

    @plsc.parallel_loop(0, COLCH, step=L)
    def _p2(b):
        acc = colchunk_v[pl.ds(b, L)]
        for l in range(1, NS):
            acc = acc + colchunk_v[pl.ds(l * COLCH + b, L)]
        combchunk_v[pl.ds(b, L)] = acc

    pltpu.sync_copy(combchunk_v, comb_sh.at[pl.ds(s * COLCH, COLCH)])
    plsc.subcore_barrier()
    pltpu.sync_copy(comb_sh, hist_v)
    scope2.__exit__(None, None, None)

    @plsc.parallel_loop(0, PLY, step=L, unroll=4)
    def _p3(b):
        den_r = 1.0 / jnp.maximum(hist_v[pl.ds(3 * PLY + b, L)], 1.0)
        hist_v[pl.ds(b, L)] = hist_v[pl.ds(b, L)] * den_r
        hist_v[pl.ds(PLY + b, L)] = hist_v[pl.ds(PLY + b, L)] * den_r
        hist_v[pl.ds(2 * PLY + b, L)] = hist_v[pl.ds(2 * PLY + b, L)] * den_r

    def phase4(base, lb, npts, unroll):
        @plsc.parallel_loop(0, npts, step=L, unroll=unroll)
        def _p4(b):
            cl = clus_v[pl.ds(lb + b, L)]
            dx = fx_v[pl.ds(lb + b, L)] - plsc.load_gather(hist_v, [cl])
            dy = fy_v[pl.ds(lb + b, L)] - plsc.load_gather(hist_v, [cl + PLY])
            dz = fz_v[pl.ds(lb + b, L)] - plsc.load_gather(hist_v,
                                                           [cl + 2 * PLY])
            ss = dx * dx + dy * dy + dz * dz
            out_v[pl.ds(b, L)] = ss * _rsqrt(jnp.maximum(ss, 1e-30))

        pltpu.sync_copy(out_v.at[pl.ds(0, npts)],
                        out_hbm.at[pl.ds(base + lb, npts)])

    with jax.named_scope("p4"):
        pl.when(s < NS - 1)(lambda: phase4(s * CHUNK1, c * HALF1, HALF1, 4))
        pl.when(s == NS - 1)(lambda: phase4((NS - 1) * CHUNK1,
                                            c * HALF1_LAST, HALF1_LAST, 5))


@jax.jit
def kernel(flow, clusters):
    fx = flow[:, 0]
    fy = flow[:, 1]
    fz = flow[:, 2]

    mesh = plsc.VectorSubcoreMesh(core_axis_name="c", subcore_axis_name="s")
    run = pl.kernel(
        _floxels_kernel,
        mesh=mesh,
        compiler_params=pltpu.CompilerParams(needs_layout_passes=False),
        out_type=jax.ShapeDtypeStruct((N_POINTS,), jnp.float32),
        scratch_types=[
            pltpu.VMEM((CHUNK1,), jnp.float32),
            pltpu.VMEM((CHUNK1,), jnp.float32),
            pltpu.VMEM((CHUNK1,), jnp.float32),
            pltpu.VMEM((CHUNK1,), jnp.int32),
            pltpu.VMEM((ACC,), jnp.float32),
            pltpu.VMEM((NS * COLCH,), jnp.float32),
            pltpu.VMEM((COLCH,), jnp.float32),
            pltpu.VMEM((HALF1,), jnp.float32),
            pltpu.SemaphoreType.DMA,
            pltpu.SemaphoreType.DMA,
            pltpu.VMEM_SHARED((NS * ACC,), jnp.float32),
            pltpu.VMEM_SHARED((ACC,), jnp.float32),
        ],
    )
    return run(fx, fy, fz, clusters)


# speedup vs baseline: 1.0073x; 1.0073x over previous
import jax
import jax.numpy as jnp
from jax import lax
from jax.experimental import pallas as pl
from jax.experimental.pallas import tpu as pltpu
from jax.experimental.pallas import tpu_sc as plsc

N_POINTS = 100000
N_BINS = 512
NC = 2
NS = 16
L = 16

CHUNK1 = 6272
CHUNK1_LAST = N_POINTS - (NS - 1) * CHUNK1
P1_CHUNKS = ((768, 4), (2752, 4), (2752, 4))
P1_CHUNKS_LAST = ((640, 4), (2640, 5), (2640, 5))
HALF1 = CHUNK1 // 2
HALF1_LAST = CHUNK1_LAST // 2
NBP = 576
ACC = 4 * NBP
PLY = NBP
COLCH = ACC // NS


def _rsqrt(x):
    xb = plsc.bitcast(x, jnp.int32)
    y = plsc.bitcast(jnp.int32(0x5F3759DF) - lax.shift_right_logical(xb, 1),
                     jnp.float32)
    hx = x * 0.5
    for _ in range(3):
        y = y * (1.5 - hx * y * y)
    return y


def _floxels_kernel(fx_hbm, fy_hbm, fz_hbm, clus_hbm, out_hbm,
                    fx_v, fy_v, fz_v, clus_v, hist_v, colchunk_v,
                    combchunk_v, out_v, sem_a, sem_b, part_sh, comb_sh):
    c = lax.axis_index("c")
    s = lax.axis_index("s")
    zeros16 = jnp.zeros((L,), jnp.float32)
    ones16 = jnp.ones((L,), jnp.float32)

    def fire(base, npts, lb, sem):
        return [
            pltpu.async_copy(fx_hbm.at[pl.ds(base, npts)],
                             fx_v.at[pl.ds(lb, npts)], sem),
            pltpu.async_copy(fy_hbm.at[pl.ds(base, npts)],
                             fy_v.at[pl.ds(lb, npts)], sem),
            pltpu.async_copy(fz_hbm.at[pl.ds(base, npts)],
                             fz_v.at[pl.ds(lb, npts)], sem),
            pltpu.async_copy(clus_hbm.at[pl.ds(base, npts)],
                             clus_v.at[pl.ds(lb, npts)], sem),
        ]

    def p1_loop(lb, npts, unroll):
        @plsc.parallel_loop(lb, lb + npts, step=L, unroll=unroll)
        def _p1(b):
            cl = clus_v[pl.ds(b, L)]
            plsc.addupdate_scatter(hist_v, [cl], fx_v[pl.ds(b, L)])
            plsc.addupdate_scatter(hist_v, [cl + PLY], fy_v[pl.ds(b, L)])
            plsc.addupdate_scatter(hist_v, [cl + 2 * PLY], fz_v[pl.ds(b, L)])
            plsc.addupdate_scatter(hist_v, [cl + 3 * PLY], ones16)

    def phase1(base, chunks):
        (sz0, u0), (sz1, u1), (sz2, u2) = chunks
        cps0 = fire(base, sz0, 0, sem_a)
        cps1 = fire(base + sz0, sz1, sz0, sem_b)

        @plsc.parallel_loop(0, ACC, step=L, unroll=4)
        def _zero(b):
            hist_v[pl.ds(b, L)] = zeros16

        for cp in cps0:
            cp.wait()
        cps2 = fire(base + sz0 + sz1, sz2, sz0 + sz1, sem_a)
        p1_loop(0, sz0, u0)
        for cp in cps1:
            cp.wait()
        p1_loop(sz0, sz1, u1)
        for cp in cps2:
            cp.wait()
        p1_loop(sz0 + sz1, sz2, u2)

    with jax.named_scope("p1"):
        pl.when(s < NS - 1)(lambda: phase1(s * CHUNK1, P1_CHUNKS))
        pl.when(s == NS - 1)(lambda: phase1((NS - 1) * CHUNK1, P1_CHUNKS_LAST))

    scope2 = jax.named_scope("p2")
    scope2.__enter__()
    pltpu.sync_copy(hist_v, part_sh.at[pl.ds(s * ACC, ACC)])
    plsc.subcore_barrier()
    cps = [
        pltpu.async_copy(part_sh.at[pl.ds(l * ACC + s * COLCH, COLCH)],
                         colchunk_v.at[pl.ds(l * COLCH, COLCH)], sem_a)
        for l in range(NS)
    ]
    for cp in cps:
        cp.wait()

    @plsc.parallel_loop(0, COLCH, step=L)
    def _p2(b):
        acc = colchunk_v[pl.ds(b, L)]
        for l in range(1, NS):
            acc = acc + colchunk_v[pl.ds(l * COLCH + b, L)]
        combchunk_v[pl.ds(b, L)] = acc

    pltpu.sync_copy(combchunk_v, comb_sh.at[pl.ds(s * COLCH, COLCH)])
    plsc.subcore_barrier()
    pltpu.sync_copy(comb_sh, hist_v)
    scope2.__exit__(None, None, None)

    @plsc.parallel_loop(0, PLY, step=L, unroll=4)
    def _p3(b):
        den_r = 1.0 / jnp.maximum(hist_v[pl.ds(3 * PLY + b, L)], 1.0)
        hist_v[pl.ds(b, L)] = hist_v[pl.ds(b, L)] * den_r
        hist_v[pl.ds(PLY + b, L)] = hist_v[pl.ds(PLY + b, L)] * den_r
        hist_v[pl.ds(2 * PLY + b, L)] = hist_v[pl.ds(2 * PLY + b, L)] * den_r

    def phase4(base, lb, npts, unroll):
        @plsc.parallel_loop(0, npts, step=L, unroll=unroll)
        def _p4(b):
            cl = clus_v[pl.ds(lb + b, L)]
            dx = fx_v[pl.ds(lb + b, L)] - plsc.load_gather(hist_v, [cl])
            dy = fy_v[pl.ds(lb + b, L)] - plsc.load_gather(hist_v, [cl + PLY])
            dz = fz_v[pl.ds(lb + b, L)] - plsc.load_gather(hist_v,
                                                           [cl + 2 * PLY])
            ss = dx * dx + dy * dy + dz * dz
            out_v[pl.ds(b, L)] = ss * _rsqrt(jnp.maximum(ss, 1e-30))

        pltpu.sync_copy(out_v.at[pl.ds(0, npts)],
                        out_hbm.at[pl.ds(base + lb, npts)])

    with jax.named_scope("p4"):
        pl.when(s < NS - 1)(lambda: phase4(s * CHUNK1, c * HALF1, HALF1, 4))
        pl.when(s == NS - 1)(lambda: phase4((NS - 1) * CHUNK1,
                                            c * HALF1_LAST, HALF1_LAST, 5))


@jax.jit
def kernel(flow, clusters):
    fx = flow[:, 0]
    fy = flow[:, 1]
    fz = flow[:, 2]

    mesh = plsc.VectorSubcoreMesh(core_axis_name="c", subcore_axis_name="s")
    run = pl.kernel(
        _floxels_kernel,
        mesh=mesh,
        compiler_params=pltpu.CompilerParams(needs_layout_passes=False),
        out_type=jax.ShapeDtypeStruct((N_POINTS,), jnp.float32),
        scratch_types=[
            pltpu.VMEM((CHUNK1,), jnp.float32),
            pltpu.VMEM((CHUNK1,), jnp.float32),
            pltpu.VMEM((CHUNK1,), jnp.float32),
            pltpu.VMEM((CHUNK1,), jnp.int32),
            pltpu.VMEM((ACC,), jnp.float32),
            pltpu.VMEM((NS * COLCH,), jnp.float32),
            pltpu.VMEM((COLCH,), jnp.float32),
            pltpu.VMEM((HALF1,), jnp.float32),
            pltpu.SemaphoreType.DMA,
            pltpu.SemaphoreType.DMA,
            pltpu.VMEM_SHARED((NS * ACC,), jnp.float32),
            pltpu.VMEM_SHARED((ACC,), jnp.float32),
        ],
    )
    return run(fx, fy, fz, clusters)
